# baseline (device time: 473035 ns/iter reference)
import jax
import jax.numpy as jnp
from jax import lax
from jax.experimental import pallas as pl
from jax.experimental.pallas import tpu as pltpu

M = 4096
N = 4096
K = 8192

STREAMS = (
    (("z2", "z1", "y", "x"), 2560, 20),
    (("z2", "z1", "x", "y"), 1536, 12),
)
MAX_CHUNKS = max(s[2] for s in STREAMS)
N_WAVES = 5


def kernel(dy, W):
    my_x = lax.axis_index("x")
    my_y = lax.axis_index("y")
    my_z = lax.axis_index("z")
    zb0 = my_z % 2
    zb1 = my_z // 2

    bits = {"x": my_x, "y": my_y, "z1": zb0, "z2": zb1}

    def block_base(order):
        b = 0
        for j, d in enumerate(order):
            b = b + bits[d] * (256 << j)
        peer = b + (1 - 2 * my_y) * (256 << order.index("y"))
        return b, peer

    base_A, peer_A = block_base(STREAMS[0][0])
    base_B, peer_B = block_base(STREAMS[1][0])

    wA = STREAMS[0][1]
    w_sA = lax.slice(W, (0, 0), (wA, K))
    w_sB = lax.slice(W, (wA, 0), (N, K))
    dn = (((1,), (1,)), ((), ()))

    def part(base, peer, w_s):
        d = jnp.concatenate([
            lax.dynamic_slice(dy, (base, 0), (256, K)),
            lax.dynamic_slice(dy, (peer, 0), (256, K)),
        ])
        return lax.dot_general(d, w_s, dn, preferred_element_type=jnp.float32)

    pA = part(base_A, peer_A, w_sA)
    pB = part(base_B, peer_B, w_sB)

    n_flows = sum(s[2] for s in STREAMS)
    n_tids = N_WAVES * n_flows

    def body(pA_ref, pB_ref, o_ref, recv, stg,
             send_sems, recv_sems, lsems):
        x = lax.axis_index("x")
        y = lax.axis_index("y")
        z = lax.axis_index("z")
        zb0 = z % 2
        zb1 = z // 2

        B = {"x": x, "y": y, "z1": zb0, "z2": zb1}
        P = {
            "x": (1 - x, y, z),
            "y": (x, 1 - y, z),
            "z1": (x, y, z + 1 - 2 * zb0),
            "z2": (x, y, z + 2 - 4 * zb1),
        }

        bar = pltpu.get_barrier_semaphore()
        for pid in P.values():
            pl.semaphore_signal(
                bar, inc=1, device_id=pid,
                device_id_type=pl.DeviceIdType.MESH,
            )
        pl.semaphore_wait(bar, 4)

        flows = []
        fi = 0
        scol = 0
        stream_refs = (pA_ref, pB_ref)
        for si, (order, width, n_chunks) in enumerate(STREAMS):
            cw = width // n_chunks
            b = 0
            for j, d in enumerate(order):
                b = b + B[d] * (256 << j)
            bases = [b]
            for j, d in enumerate(order):
                bases.append(bases[-1] - B[d] * (256 << j))
            for c in range(n_chunks):
                flows.append({
                    "fi": fi, "ci": c, "order": order,
                    "p": stream_refs[si],
                    "c0l": c * cw, "c0g": scol + c * cw, "cw": cw,
                    "bases": bases,
                })
                fi += 1
            scol += width

        pend = {}

        def start_rdma(tid, partner, src, dst):
            rd = pltpu.make_async_remote_copy(
                src_ref=src, dst_ref=dst,
                send_sem=send_sems.at[tid], recv_sem=recv_sems.at[tid],
                device_id=partner, device_id_type=pl.DeviceIdType.MESH,
            )
            rd.start()
            return rd

        def issue(f, w):
            fi, cw, c0g = f["fi"], f["cw"], f["c0g"]
            colsg = pl.ds(c0g, cw)
            colsl = pl.ds(f["c0l"], cw)
            order, bases = f["order"], f["bases"]
            tid = N_WAVES * fi + w
            if w == 0:
                cp = pltpu.make_async_copy(
                    f["p"].at[pl.ds(0, 256), colsl],
                    stg.at[pl.ds(0, 256), colsg],
                    lsems.at[2 * fi],
                )
                cp.start()
                rd = start_rdma(
                    tid, P["y"],
                    f["p"].at[pl.ds(256, 256), colsl],
                    recv.at[pl.ds(0, 256), colsg],
                )

                def fin(rd=rd, cp=cp, c0g=c0g, cw=cw):
                    rd.wait()
                    cp.wait()
                    r = pl.ds(0, 256)
                    cg = pl.ds(c0g, cw)
                    stg[r, cg] = stg[r, cg] + recv[0:256, c0g:c0g + cw]
                pend[(fi, w)] = fin
            elif w == 1:
                cp = pltpu.make_async_copy(
                    stg.at[pl.ds(0, 256), colsg],
                    o_ref.at[pl.ds(bases[0], 256), colsg],
                    lsems.at[2 * fi + 1],
                )
                cp.start()
                rd = start_rdma(
                    tid, P[order[0]],
                    stg.at[pl.ds(0, 256), colsg],
                    o_ref.at[pl.ds(bases[0], 256), colsg],
                )

                def fin(rd=rd, cp=cp):
                    rd.wait()
                    cp.wait()
                pend[(fi, w)] = fin
            else:
                n = 256 << (w - 1)
                rd = start_rdma(
                    tid, P[order[w - 1]],
                    o_ref.at[pl.ds(bases[w - 1], n), colsg],
                    o_ref.at[pl.ds(bases[w - 1], n), colsg],
                )

                def fin(rd=rd):
                    rd.wait()
                pend[(fi, w)] = fin

        for t in range(N_WAVES + MAX_CHUNKS - 1):
            for f in flows:
                w = t - f["ci"]
                if 0 <= w < N_WAVES:
                    if w > 0:
                        pend.pop((f["fi"], w - 1))()
                    issue(f, w)
        for f in flows:
            pend.pop((f["fi"], N_WAVES - 1))()

    return pl.pallas_call(
        body,
        out_shape=jax.ShapeDtypeStruct((M, N), jnp.float32),
        in_specs=[pl.BlockSpec(memory_space=pl.ANY)] * 2,
        out_specs=pl.BlockSpec(memory_space=pl.ANY),
        scratch_shapes=[
            pltpu.VMEM((256, N), jnp.float32),
            pltpu.VMEM((256, N), jnp.float32),
            pltpu.SemaphoreType.DMA((n_tids,)),
            pltpu.SemaphoreType.DMA((n_tids,)),
            pltpu.SemaphoreType.DMA((2 * n_flows,)),
        ],
        compiler_params=pltpu.CompilerParams(
            collective_id=0,
            vmem_limit_bytes=56 * 1024 * 1024,
        ),
    )(pA, pB)
